# baseline (speedup 1.0000x reference)
import jax
import jax.numpy as jnp
from jax import lax
from jax.experimental import pallas as pl
from jax.experimental.pallas import tpu as pltpu
from jax.experimental.pallas import tpu_sc as plsc

NTOK = 1000
B, L = 1024, 50
ROWS = B * L
NC, NS = 2, 16
RPW = ROWS // (NC * NS)
ZWORDS = 800_000
NDMA = RPW * NTOK // ZWORDS
NGRP = RPW // 64
LANES = 16


def _body(x_hbm, zeros_hbm, out_hbm, zbuf, xv, idx2d, val2d, s0, s1, s2):
    cid = lax.axis_index("c")
    tid = lax.axis_index("s")
    rowbase = (cid * NS + tid) * RPW
    wordbase = rowbase * NTOK

    @pl.when(tid == 0)
    def _():
        pltpu.sync_copy(zeros_hbm, zbuf)

    plsc.subcore_barrier()

    zc0 = pltpu.async_copy(zbuf, out_hbm.at[pl.ds(wordbase, ZWORDS)], s0)
    zc1 = pltpu.async_copy(zbuf, out_hbm.at[pl.ds(wordbase + ZWORDS, ZWORDS)], s1)

    pltpu.sync_copy(x_hbm.at[pl.ds(rowbase, RPW)], xv)
    lane = lax.iota(jnp.int32, LANES)
    for j in range(NGRP):
        for k in range(4):
            g = j * 4 + k
            v = xv[pl.ds(g * LANES, LANES)]
            ok = (v >= 0) & (v < NTOK)
            pos = (wordbase + g * LANES * NTOK) + lane * NTOK + jnp.where(ok, v, 0)
            idx2d[j, pl.ds(k * LANES, LANES)] = pos
            val2d[j, pl.ds(k * LANES, LANES)] = jnp.where(ok, 1.0, 0.0)

    zc0.wait()
    zc1.wait()

    scats = [
        pltpu.async_copy(val2d.at[j], out_hbm.at[idx2d.at[j]], s2)
        for j in range(NGRP)
    ]
    for c in scats:
        c.wait()


@jax.jit
def kernel(x):
    mesh = plsc.VectorSubcoreMesh(
        core_axis_name="c", subcore_axis_name="s",
        num_cores=NC, num_subcores=NS,
    )
    run = pl.kernel(
        _body,
        out_type=jax.ShapeDtypeStruct((ROWS * NTOK,), jnp.float32),
        mesh=mesh,
        scratch_types=[
            pltpu.VMEM_SHARED((ZWORDS,), jnp.float32),
            pltpu.VMEM((RPW,), jnp.int32),
            pltpu.VMEM((NGRP, 64), jnp.int32),
            pltpu.VMEM((NGRP, 64), jnp.float32),
            pltpu.SemaphoreType.DMA,
            pltpu.SemaphoreType.DMA,
            pltpu.SemaphoreType.DMA,
        ],
        compiler_params=pltpu.CompilerParams(needs_layout_passes=False),
    )
    zeros = jnp.zeros((ZWORDS,), jnp.float32)
    flat = run(x.reshape(ROWS).astype(jnp.int32), zeros)
    return flat.reshape(B, L, NTOK)


# speedup vs baseline: 2.0423x; 2.0423x over previous
import jax
import jax.numpy as jnp
from jax import lax
from jax.experimental import pallas as pl
from jax.experimental.pallas import tpu as pltpu
from jax.experimental.pallas import tpu_sc as plsc

NTOK = 1000
B, L = 1024, 50
NC, NS = 2, 16
BPW = B // (NC * NS)
LANES = 16
NG = 4


def _body(x_hbm, out_hbm, xv, buf0, buf1, s0, s1):
    wid = lax.axis_index("c") * NS + lax.axis_index("s")
    b0 = wid * BPW

    pltpu.sync_copy(x_hbm.at[pl.ds(b0 * L, BPW * L)], xv)

    def _zero(l):
        z = jnp.zeros((LANES,), jnp.float32)
        for c in range(NTOK // LANES):
            buf0[l, pl.ds(c * LANES, LANES)] = z
            buf1[l, pl.ds(c * LANES, LANES)] = z
        buf0[l, pl.ds(NTOK - LANES, LANES)] = z
        buf1[l, pl.ds(NTOK - LANES, LANES)] = z

    pl.loop(0, L)(_zero)

    lane = lax.iota(jnp.int32, LANES)
    ones = jnp.ones((LANES,), jnp.float32)
    zeros = jnp.zeros((LANES,), jnp.float32)

    def scatter(buf, i, value):
        for l0 in (0, LANES, 2 * LANES, L - LANES):
            v = xv[pl.ds(i * L + l0, LANES)]
            ok = (v >= 0) & (v < NTOK)
            l_idx = l0 + lane
            v_idx = jnp.where(ok, v, 0)
            plsc.store_scatter(buf, [l_idx, v_idx], value, mask=ok)

    bufs = (buf0, buf1)
    sems = (s0, s1)
    copies = [None, None]
    for i in range(BPW):
        s = i % 2
        if copies[s] is not None:
            copies[s].wait()
            scatter(bufs[s], i - 2, zeros)
        scatter(bufs[s], i, ones)
        copies[s] = pltpu.async_copy(bufs[s], out_hbm.at[b0 + i], sems[s])
    copies[(BPW - 1) % 2].wait()
    copies[BPW % 2].wait()


@jax.jit
def kernel(x):
    mesh = plsc.VectorSubcoreMesh(
        core_axis_name="c", subcore_axis_name="s",
        num_cores=NC, num_subcores=NS,
    )
    run = pl.kernel(
        _body,
        out_type=jax.ShapeDtypeStruct((B, L, NTOK), jnp.float32),
        mesh=mesh,
        scratch_types=[
            pltpu.VMEM((BPW * L,), jnp.int32),
            pltpu.VMEM((L, NTOK), jnp.float32),
            pltpu.VMEM((L, NTOK), jnp.float32),
            pltpu.SemaphoreType.DMA,
            pltpu.SemaphoreType.DMA,
        ],
        compiler_params=pltpu.CompilerParams(
            needs_layout_passes=False,
            use_tc_tiling_on_sc=True,
        ),
    )
    return run(x.reshape(B * L).astype(jnp.int32))
